# baseline (speedup 1.0000x reference)
import functools

import jax
import jax.numpy as jnp
from jax import lax
from jax.experimental import pallas as pl
from jax.experimental.pallas import tpu as pltpu
from jax.experimental.pallas import tpu_sc as plsc

_TOP_K = 8
_E = 64
_LANES = 16


def _matmul_body(x_ref, w_ref, lg_ref):
    lg_ref[...] = jnp.dot(x_ref[...], w_ref[...],
                          preferred_element_type=jnp.float32)


@functools.partial(jax.jit, static_argnames=("block_t",))
def _logits_tc(x, W_gate, block_t=1024):
    T, D = x.shape
    E = W_gate.shape[1]
    nb = T // block_t
    return pl.pallas_call(
        _matmul_body,
        grid=(nb,),
        in_specs=[
            pl.BlockSpec((block_t, D), lambda i: (i, 0)),
            pl.BlockSpec((D, E), lambda i: (0, 0)),
        ],
        out_specs=pl.BlockSpec((block_t, E), lambda i: (i, 0)),
        out_shape=jax.ShapeDtypeStruct((T, E), jnp.float32),
        compiler_params=pltpu.CompilerParams(
            dimension_semantics=("arbitrary",),
        ),
    )(x, W_gate)


def _merge_desc(ka, va, kb, vb):
    rkb = lax.rev(kb, (0,))
    rvb = lax.rev(vb, (0,))
    p = ka >= rkb
    hk = jnp.where(p, ka, rkb)
    hv = jnp.where(p, va, rvb)
    return plsc.sort_key_val(hk, hv, descending=True)


def _make_sc_topk(T, rows_per_chunk):
    n_workers = 32
    rows_per_worker = T // n_workers
    n_chunks = rows_per_worker // rows_per_chunk
    groups_per_chunk = rows_per_chunk // _LANES
    mesh = plsc.VectorSubcoreMesh(core_axis_name="c", subcore_axis_name="s")

    @functools.partial(
        pl.kernel,
        mesh=mesh,
        out_type=[
            jax.ShapeDtypeStruct((T * _E,), jnp.float32),
            jax.ShapeDtypeStruct((T * _TOP_K,), jnp.int32),
        ],
        scratch_types=[
            pltpu.VMEM((rows_per_chunk * _E,), jnp.float32),
            pltpu.VMEM((rows_per_chunk * _E,), jnp.float32),
            pltpu.VMEM((rows_per_chunk * _TOP_K,), jnp.int32),
        ],
        compiler_params=pltpu.CompilerParams(needs_layout_passes=False),
    )
    def sc_topk(lg_hbm, dense_hbm, idx_hbm, lg_v, dense_v, idx_v):
        wid = lax.axis_index("s") * 2 + lax.axis_index("c")
        lane = lax.broadcasted_iota(jnp.int32, (_LANES,), 0)
        low8 = lane < _TOP_K
        zeros16 = jnp.zeros((_LANES,), jnp.float32)

        def do_chunk(ci, carry):
            base = wid * rows_per_worker + ci * rows_per_chunk
            pltpu.sync_copy(lg_hbm.at[pl.ds(base * _E, rows_per_chunk * _E)],
                            lg_v)

            def zero_row(j, c):
                dense_v[pl.ds(j * _LANES, _LANES)] = zeros16
                return c

            lax.fori_loop(0, rows_per_chunk * _E // _LANES, zero_row, 0)

            def do_row(r, c):
                off = r * _E
                ks = []
                vs = []
                for cc in range(4):
                    k = lg_v[pl.ds(off + cc * _LANES, _LANES)]
                    v = lane + cc * _LANES
                    ks_c, vs_c = plsc.sort_key_val(k, v, descending=True)
                    ks.append(ks_c)
                    vs.append(vs_c)
                k01, v01 = _merge_desc(ks[0], vs[0], ks[1], vs[1])
                k23, v23 = _merge_desc(ks[2], vs[2], ks[3], vs[3])
                kf, vf = _merge_desc(k01, v01, k23, v23)
                t0 = lax.reduce_max(kf, (0,))
                e = jnp.where(low8, jnp.exp(kf - t0), 0.0)
                s = lax.reduce_sum(e, (0,))
                gates = e / s
                plsc.store_scatter(dense_v, [off + vf], gates, mask=low8)
                plsc.store_scatter(idx_v, [r * _TOP_K + lane], vf, mask=low8)
                return c

            lax.fori_loop(0, rows_per_chunk, do_row, 0)
            pltpu.sync_copy(dense_v,
                            dense_hbm.at[pl.ds(base * _E,
                                               rows_per_chunk * _E)])
            pltpu.sync_copy(idx_v,
                            idx_hbm.at[pl.ds(base * _TOP_K,
                                             rows_per_chunk * _TOP_K)])
            return carry

        lax.fori_loop(0, n_chunks, do_chunk, 0)

    return sc_topk


@jax.jit
def _router(x, W_gate):
    T = x.shape[0]
    E = W_gate.shape[1]
    logits = _logits_tc(x, W_gate)
    sc = _make_sc_topk(T, 512)
    dense_flat, idx_flat = sc(logits.reshape(T * E))
    return dense_flat.reshape(T, E), idx_flat.reshape(T, _TOP_K)


def kernel(x, W_gate):
    dense_gates, topk_idx = _router(x, W_gate)
    return dense_gates, topk_idx
